# Initial kernel scaffold; baseline (speedup 1.0000x reference)
import jax
import jax.numpy as jnp
from jax import lax
from jax.experimental import pallas as pl
from jax.experimental.pallas import tpu as pltpu
from jax.experimental.pallas import tpu_sc as plsc

N = 10000
D = 128
HID = 128
C = 16
K = 10
E = 320000

NTILES = 32
BLK = 128
NBLK = 79
EPT = NBLK * BLK
E_PAD = NTILES * EPT
RPT = N // 16
NP2 = 10112

_mesh = plsc.VectorSubcoreMesh(core_axis_name="c", subcore_axis_name="s")


@pl.kernel(
    out_type=jax.ShapeDtypeStruct((2, N, C), jnp.float32),
    mesh=_mesh,
    scratch_types=[
        pltpu.VMEM((NBLK, BLK), jnp.int32),
        pltpu.VMEM((NBLK, BLK), jnp.int32),
        pltpu.VMEM((BLK, C), jnp.float32),
        pltpu.VMEM_SHARED((NP2, C), jnp.float32),
    ],
)
def _propagate(v_hbm, zer_hbm, row_hbm, col_hbm, out_hbm, row_v, col_v, gbuf, acc):
    c = lax.axis_index("c")
    s = lax.axis_index("s")
    tid = s * 2 + c
    pltpu.sync_copy(row_hbm.at[tid], row_v)
    pltpu.sync_copy(col_hbm.at[tid], col_v)

    r0 = s * RPT

    @pl.when(c == 0)
    def _():
        pltpu.sync_copy(v_hbm.at[pl.ds(r0, RPT)], acc.at[pl.ds(r0, RPT)])

    @pl.when(c != 0)
    def _():
        pltpu.sync_copy(zer_hbm.at[pl.ds(r0, RPT)], acc.at[pl.ds(r0, RPT)])

    plsc.subcore_barrier()

    @pl.loop(0, NBLK)
    def _(j):
        pltpu.sync_copy(v_hbm.at[row_v.at[j]], gbuf)
        pltpu.sync_copy(gbuf, acc.at[col_v.at[j]], add=True)

    plsc.subcore_barrier()
    pltpu.sync_copy(acc.at[pl.ds(r0, RPT)], out_hbm.at[c, pl.ds(r0, RPT)])


def _mlp_body(x_ref, w1_ref, b1_ref, w2_ref, b2_ref, o_ref):
    h1 = jnp.dot(x_ref[...], w1_ref[...], preferred_element_type=jnp.float32)
    h1 = jnp.maximum(h1 + b1_ref[...], 0.0)
    o_ref[...] = jnp.dot(h1, w2_ref[...], preferred_element_type=jnp.float32) + b2_ref[...]


def _prep_body(s0_ref, s1_ref, h_ref, t0_ref, s_o, di_o, v_o, hid_o):
    deg = s0_ref[...] + s1_ref[...]
    sc = lax.rsqrt(deg)
    h = h_ref[...]
    s_o[...] = sc
    di_o[...] = 1.0 / deg
    v_o[...] = h * sc
    hid_o[...] = (t0_ref[0, 0] * deg) * sc * h


def _acc_body(s0_ref, s1_ref, di_ref, hid_ref, tk_ref, v_o, hid_o):
    u = s0_ref[...] + s1_ref[...]
    hid_o[...] = hid_ref[...] + tk_ref[0, 0] * u
    v_o[...] = u * di_ref[...]


def _final_body(hid_ref, s_ref, o_ref):
    hd = hid_ref[...] * s_ref[...]
    m = jnp.max(hd, axis=1, keepdims=True)
    lse = jnp.log(jnp.sum(jnp.exp(hd - m), axis=1, keepdims=True))
    o_ref[...] = hd - m - lse


_NC_F32 = jax.ShapeDtypeStruct((N, C), jnp.float32)


def kernel(x, edge_index, W1, b1, W2, b2, temp):
    row = edge_index[0]
    col = edge_index[1]
    pad = E_PAD - E
    rowp = jnp.concatenate([row, jnp.zeros((pad,), row.dtype)])
    colp = jnp.concatenate([col, jnp.full((pad,), N, col.dtype)])
    row3 = rowp.reshape(NTILES, NBLK, BLK)
    col3 = colp.reshape(NTILES, NBLK, BLK)

    zer = jnp.zeros((N, C), jnp.float32)
    ones = jnp.ones((N, C), jnp.float32)

    h = pl.pallas_call(_mlp_body, out_shape=_NC_F32)(
        x, W1, b1.reshape(1, HID), W2, b2.reshape(1, C))

    sdeg = _propagate(ones, zer, row3, col3)

    s, di, v, hid = pl.pallas_call(
        _prep_body, out_shape=(_NC_F32, _NC_F32, _NC_F32, _NC_F32))(
        sdeg[0], sdeg[1], h, temp[0].reshape(1, 1))

    for k in range(K):
        part = _propagate(v, zer, row3, col3)
        v, hid = pl.pallas_call(_acc_body, out_shape=(_NC_F32, _NC_F32))(
            part[0], part[1], di, hid, temp[k + 1].reshape(1, 1))

    return pl.pallas_call(_final_body, out_shape=_NC_F32)(hid, s)


# speedup vs baseline: 15.9894x; 15.9894x over previous
import jax
import jax.numpy as jnp
from jax import lax
from jax.experimental import pallas as pl
from jax.experimental.pallas import tpu as pltpu
from jax.experimental.pallas import tpu_sc as plsc

N = 10000
D = 128
HID = 128
C = 16
K = 10
E = 320000

NTILES = 32
BLK = 128
NBLK = 80
EPT = NBLK * BLK
E_PAD = NTILES * EPT
NP = 10112
RPT = NP // 16

_mesh = plsc.VectorSubcoreMesh(core_axis_name="c", subcore_axis_name="s")


@pl.kernel(
    out_type=jax.ShapeDtypeStruct((2, NP, C), jnp.float32),
    mesh=_mesh,
    scratch_types=[
        pltpu.VMEM((NBLK, BLK), jnp.int32),
        pltpu.VMEM((NBLK, BLK), jnp.int32),
        pltpu.VMEM((BLK, C), jnp.float32),
        pltpu.VMEM_SHARED((NP, C), jnp.float32),
    ],
    compiler_params=pltpu.CompilerParams(use_tc_tiling_on_sc=False),
)
def _propagate(v_hbm, zer_hbm, row_hbm, col_hbm, out_hbm, row_v, col_v, gbuf, acc):
    c = lax.axis_index("c")
    s = lax.axis_index("s")
    tid = s * 2 + c
    pltpu.sync_copy(row_hbm.at[tid], row_v)
    pltpu.sync_copy(col_hbm.at[tid], col_v)

    r0 = s * RPT

    @pl.when(c == 0)
    def _():
        pltpu.sync_copy(v_hbm.at[pl.ds(r0, RPT)], acc.at[pl.ds(r0, RPT)])

    @pl.when(c != 0)
    def _():
        pltpu.sync_copy(zer_hbm.at[pl.ds(r0, RPT)], acc.at[pl.ds(r0, RPT)])

    plsc.subcore_barrier()

    @pl.loop(0, NBLK)
    def _(j):
        pltpu.sync_copy(v_hbm.at[row_v.at[j]], gbuf)
        pltpu.sync_copy(gbuf, acc.at[col_v.at[j]], add=True)

    plsc.subcore_barrier()
    pltpu.sync_copy(acc.at[pl.ds(r0, RPT)], out_hbm.at[c, pl.ds(r0, RPT)])


def _mlp_body(x_ref, w1_ref, b1_ref, w2_ref, b2_ref, o_ref):
    h1 = jnp.dot(x_ref[...], w1_ref[...], preferred_element_type=jnp.float32)
    h1 = jnp.maximum(h1 + b1_ref[...], 0.0)
    o_ref[:N, :] = jnp.dot(h1, w2_ref[...], preferred_element_type=jnp.float32) + b2_ref[...]
    o_ref[N:, :] = jnp.zeros((NP - N, C), jnp.float32)


def _prep_body(s0_ref, s1_ref, h_ref, t0_ref, s_o, di_o, v_o, hid_o):
    deg = s0_ref[...] + s1_ref[...]
    sc = lax.rsqrt(deg)
    h = h_ref[...]
    s_o[...] = sc
    di_o[...] = 1.0 / deg
    v_o[...] = h * sc
    hid_o[...] = (t0_ref[0, 0] * deg) * sc * h


def _acc_body(s0_ref, s1_ref, di_ref, hid_ref, tk_ref, v_o, hid_o):
    u = s0_ref[...] + s1_ref[...]
    hid_o[...] = hid_ref[...] + tk_ref[0, 0] * u
    v_o[...] = u * di_ref[...]


def _final_body(hid_ref, s_ref, o_ref):
    hd = hid_ref[:N, :] * s_ref[:N, :]
    m = jnp.max(hd, axis=1, keepdims=True)
    lse = jnp.log(jnp.sum(jnp.exp(hd - m), axis=1, keepdims=True))
    o_ref[...] = hd - m - lse


_NP_F32 = jax.ShapeDtypeStruct((NP, C), jnp.float32)


def kernel(x, edge_index, W1, b1, W2, b2, temp):
    row = edge_index[0]
    col = edge_index[1]
    pad = E_PAD - E
    rowp = jnp.concatenate([row, jnp.zeros((pad,), row.dtype)])
    colp = jnp.concatenate([col, jnp.full((pad,), N, col.dtype)])
    row3 = rowp.reshape(NTILES, NBLK, BLK)
    col3 = colp.reshape(NTILES, NBLK, BLK)

    zer = jnp.zeros((NP, C), jnp.float32)
    ones = jnp.ones((NP, C), jnp.float32)

    h = pl.pallas_call(_mlp_body, out_shape=_NP_F32)(
        x, W1, b1.reshape(1, HID), W2, b2.reshape(1, C))

    sdeg = _propagate(ones, zer, row3, col3)

    s, di, v, hid = pl.pallas_call(
        _prep_body, out_shape=(_NP_F32, _NP_F32, _NP_F32, _NP_F32))(
        sdeg[0], sdeg[1], h, temp[0].reshape(1, 1))

    for k in range(K):
        part = _propagate(v, zer, row3, col3)
        v, hid = pl.pallas_call(_acc_body, out_shape=(_NP_F32, _NP_F32))(
            part[0], part[1], di, hid, temp[k + 1].reshape(1, 1))

    return pl.pallas_call(_final_body, out_shape=jax.ShapeDtypeStruct((N, C), jnp.float32))(hid, s)


# speedup vs baseline: 21.7453x; 1.3600x over previous
import jax
import jax.numpy as jnp
from jax import lax
from jax.experimental import pallas as pl
from jax.experimental.pallas import tpu as pltpu
from jax.experimental.pallas import tpu_sc as plsc

N = 10000
D = 128
HID = 128
C = 16
K = 10
E = 320000

NTILES = 32
BLK = 128
NBLK = 80
EPT = NBLK * BLK
E_PAD = NTILES * EPT
NP = 10112
RPT = NP // 16
NBUF = 8
NGRP = NBLK // NBUF

_mesh = plsc.VectorSubcoreMesh(core_axis_name="c", subcore_axis_name="s")


@pl.kernel(
    out_type=jax.ShapeDtypeStruct((2, NP, C), jnp.float32),
    mesh=_mesh,
    scratch_types=[
        pltpu.VMEM((NBLK, BLK), jnp.int32),
        pltpu.VMEM((NBLK, BLK), jnp.int32),
        pltpu.VMEM((NBUF, BLK, C), jnp.float32),
        pltpu.VMEM_SHARED((NP, C), jnp.float32),
        pltpu.SemaphoreType.DMA((NBUF,)),
        pltpu.SemaphoreType.DMA((NBUF,)),
    ],
    compiler_params=pltpu.CompilerParams(use_tc_tiling_on_sc=False),
)
def _propagate(v_hbm, zer_hbm, row_hbm, col_hbm, out_hbm,
               row_v, col_v, gbuf, acc, semg, sems):
    c = lax.axis_index("c")
    s = lax.axis_index("s")
    tid = s * 2 + c
    pltpu.sync_copy(row_hbm.at[tid], row_v)
    pltpu.sync_copy(col_hbm.at[tid], col_v)

    r0 = s * RPT

    @pl.when(c == 0)
    def _():
        pltpu.sync_copy(v_hbm.at[pl.ds(r0, RPT)], acc.at[pl.ds(r0, RPT)])

    @pl.when(c != 0)
    def _():
        pltpu.sync_copy(zer_hbm.at[pl.ds(r0, RPT)], acc.at[pl.ds(r0, RPT)])

    plsc.subcore_barrier()

    def _gather(j, b):
        pltpu.async_copy(v_hbm.at[row_v.at[j]], gbuf.at[b], semg.at[b])

    def _gather_wait(j, b):
        pltpu.make_async_copy(v_hbm.at[row_v.at[j]], gbuf.at[b], semg.at[b]).wait()

    def _scatter(j, b):
        pltpu.async_copy(gbuf.at[b], acc.at[col_v.at[j]], sems.at[b], add=True)

    def _scatter_wait(j, b):
        pltpu.make_async_copy(gbuf.at[b], acc.at[col_v.at[j]], sems.at[b]).wait()

    for b in range(NBUF):
        _gather(b, b)

    @pl.loop(0, NGRP - 1)
    def _(g):
        cur = g * NBUF
        nxt = cur + NBUF
        for b in range(NBUF):
            _gather_wait(cur + b, b)
            _scatter(cur + b, b)
        for b in range(NBUF):
            _scatter_wait(cur + b, b)
            _gather(nxt + b, b)

    last = (NGRP - 1) * NBUF
    for b in range(NBUF):
        _gather_wait(last + b, b)
        _scatter(last + b, b)
    for b in range(NBUF):
        _scatter_wait(last + b, b)

    plsc.subcore_barrier()
    pltpu.sync_copy(acc.at[pl.ds(r0, RPT)], out_hbm.at[c, pl.ds(r0, RPT)])


def _mlp_body(x_ref, w1_ref, b1_ref, w2_ref, b2_ref, o_ref):
    h1 = jnp.dot(x_ref[...], w1_ref[...], preferred_element_type=jnp.float32)
    h1 = jnp.maximum(h1 + b1_ref[...], 0.0)
    o_ref[:N, :] = jnp.dot(h1, w2_ref[...], preferred_element_type=jnp.float32) + b2_ref[...]
    o_ref[N:, :] = jnp.zeros((NP - N, C), jnp.float32)


def _prep_body(s0_ref, s1_ref, h_ref, t0_ref, s_o, di_o, v_o, hid_o):
    deg = s0_ref[...] + s1_ref[...]
    sc = lax.rsqrt(deg)
    h = h_ref[...]
    s_o[...] = sc
    di_o[...] = 1.0 / deg
    v_o[...] = h * sc
    hid_o[...] = (t0_ref[0, 0] * deg) * sc * h


def _acc_body(s0_ref, s1_ref, di_ref, hid_ref, tk_ref, v_o, hid_o):
    u = s0_ref[...] + s1_ref[...]
    hid_o[...] = hid_ref[...] + tk_ref[0, 0] * u
    v_o[...] = u * di_ref[...]


def _final_body(hid_ref, s_ref, o_ref):
    hd = hid_ref[:N, :] * s_ref[:N, :]
    m = jnp.max(hd, axis=1, keepdims=True)
    lse = jnp.log(jnp.sum(jnp.exp(hd - m), axis=1, keepdims=True))
    o_ref[...] = hd - m - lse


_NP_F32 = jax.ShapeDtypeStruct((NP, C), jnp.float32)


def kernel(x, edge_index, W1, b1, W2, b2, temp):
    row = edge_index[0]
    col = edge_index[1]
    pad = E_PAD - E
    rowp = jnp.concatenate([row, jnp.zeros((pad,), row.dtype)])
    colp = jnp.concatenate([col, jnp.full((pad,), N, col.dtype)])
    row3 = rowp.reshape(NTILES, NBLK, BLK)
    col3 = colp.reshape(NTILES, NBLK, BLK)

    zer = jnp.zeros((NP, C), jnp.float32)
    ones = jnp.ones((NP, C), jnp.float32)

    h = pl.pallas_call(_mlp_body, out_shape=_NP_F32)(
        x, W1, b1.reshape(1, HID), W2, b2.reshape(1, C))

    sdeg = _propagate(ones, zer, row3, col3)

    s, di, v, hid = pl.pallas_call(
        _prep_body, out_shape=(_NP_F32, _NP_F32, _NP_F32, _NP_F32))(
        sdeg[0], sdeg[1], h, temp[0].reshape(1, 1))

    for k in range(K):
        part = _propagate(v, zer, row3, col3)
        v, hid = pl.pallas_call(_acc_body, out_shape=(_NP_F32, _NP_F32))(
            part[0], part[1], di, hid, temp[k + 1].reshape(1, 1))

    return pl.pallas_call(_final_body, out_shape=jax.ShapeDtypeStruct((N, C), jnp.float32))(hid, s)


# speedup vs baseline: 35.5753x; 1.6360x over previous
import jax
import jax.numpy as jnp
from jax import lax
from jax.experimental import pallas as pl
from jax.experimental.pallas import tpu as pltpu
from jax.experimental.pallas import tpu_sc as plsc

N = 10000
D = 128
HID = 128
C = 16
K = 10
E = 320000

NTILES = 32
BLK = 128
NBLK = 80
EPT = NBLK * BLK
E_PAD = NTILES * EPT
NP = 10112
RPT = NP // 16
NBUF = 8
NGRP = NBLK // NBUF

_mesh = plsc.VectorSubcoreMesh(core_axis_name="c", subcore_axis_name="s")


@pl.kernel(
    out_type=jax.ShapeDtypeStruct((2, NP, C), jnp.float32),
    mesh=_mesh,
    scratch_types=[
        pltpu.VMEM((NBLK, BLK), jnp.int32),
        pltpu.VMEM((NBLK, BLK), jnp.int32),
        pltpu.VMEM((NBUF, BLK, C), jnp.float32),
        pltpu.VMEM_SHARED((NP, C), jnp.float32),
        pltpu.VMEM_SHARED((NP, C), jnp.float32),
        pltpu.SemaphoreType.DMA((NBUF,)),
        pltpu.SemaphoreType.DMA((NBUF,)),
    ],
    compiler_params=pltpu.CompilerParams(use_tc_tiling_on_sc=False),
)
def _propagate(v_hbm, zer_hbm, row_hbm, col_hbm, out_hbm,
               row_v, col_v, gbuf, acc, vsh, semg, sems):
    c = lax.axis_index("c")
    s = lax.axis_index("s")
    tid = s * 2 + c
    pltpu.sync_copy(row_hbm.at[tid], row_v)
    pltpu.sync_copy(col_hbm.at[tid], col_v)

    r0 = s * RPT
    pltpu.sync_copy(v_hbm.at[pl.ds(r0, RPT)], vsh.at[pl.ds(r0, RPT)])

    @pl.when(c == 0)
    def _():
        pltpu.sync_copy(v_hbm.at[pl.ds(r0, RPT)], acc.at[pl.ds(r0, RPT)])

    @pl.when(c != 0)
    def _():
        pltpu.sync_copy(zer_hbm.at[pl.ds(r0, RPT)], acc.at[pl.ds(r0, RPT)])

    plsc.subcore_barrier()

    def _gather(j, b):
        pltpu.async_copy(vsh.at[row_v.at[j]], gbuf.at[b], semg.at[b])

    def _gather_wait(j, b):
        pltpu.make_async_copy(vsh.at[row_v.at[j]], gbuf.at[b], semg.at[b]).wait()

    def _scatter(j, b):
        pltpu.async_copy(gbuf.at[b], acc.at[col_v.at[j]], sems.at[b], add=True)

    def _scatter_wait(j, b):
        pltpu.make_async_copy(gbuf.at[b], acc.at[col_v.at[j]], sems.at[b]).wait()

    for b in range(NBUF):
        _gather(b, b)

    @pl.loop(0, NGRP - 1)
    def _(g):
        cur = g * NBUF
        nxt = cur + NBUF
        for b in range(NBUF):
            _gather_wait(cur + b, b)
            _scatter(cur + b, b)
        for b in range(NBUF):
            _scatter_wait(cur + b, b)
            _gather(nxt + b, b)

    last = (NGRP - 1) * NBUF
    for b in range(NBUF):
        _gather_wait(last + b, b)
        _scatter(last + b, b)
    for b in range(NBUF):
        _scatter_wait(last + b, b)

    plsc.subcore_barrier()
    pltpu.sync_copy(acc.at[pl.ds(r0, RPT)], out_hbm.at[c, pl.ds(r0, RPT)])


def _mlp_body(x_ref, w1_ref, b1_ref, w2_ref, b2_ref, o_ref):
    h1 = jnp.dot(x_ref[...], w1_ref[...], preferred_element_type=jnp.float32)
    h1 = jnp.maximum(h1 + b1_ref[...], 0.0)
    o_ref[:N, :] = jnp.dot(h1, w2_ref[...], preferred_element_type=jnp.float32) + b2_ref[...]
    o_ref[N:, :] = jnp.zeros((NP - N, C), jnp.float32)


def _prep_body(s0_ref, s1_ref, h_ref, t0_ref, s_o, di_o, v_o, hid_o):
    deg = s0_ref[...] + s1_ref[...]
    sc = lax.rsqrt(deg)
    h = h_ref[...]
    s_o[...] = sc
    di_o[...] = 1.0 / deg
    v_o[...] = h * sc
    hid_o[...] = (t0_ref[0, 0] * deg) * sc * h


def _acc_body(s0_ref, s1_ref, di_ref, hid_ref, tk_ref, v_o, hid_o):
    u = s0_ref[...] + s1_ref[...]
    hid_o[...] = hid_ref[...] + tk_ref[0, 0] * u
    v_o[...] = u * di_ref[...]


def _final_body(hid_ref, s_ref, o_ref):
    hd = hid_ref[:N, :] * s_ref[:N, :]
    m = jnp.max(hd, axis=1, keepdims=True)
    lse = jnp.log(jnp.sum(jnp.exp(hd - m), axis=1, keepdims=True))
    o_ref[...] = hd - m - lse


_NP_F32 = jax.ShapeDtypeStruct((NP, C), jnp.float32)


def kernel(x, edge_index, W1, b1, W2, b2, temp):
    row = edge_index[0]
    col = edge_index[1]
    pad = E_PAD - E
    rowp = jnp.concatenate([row, jnp.zeros((pad,), row.dtype)])
    colp = jnp.concatenate([col, jnp.full((pad,), N, col.dtype)])
    row3 = rowp.reshape(NTILES, NBLK, BLK)
    col3 = colp.reshape(NTILES, NBLK, BLK)

    zer = jnp.zeros((NP, C), jnp.float32)
    ones = jnp.ones((NP, C), jnp.float32)

    h = pl.pallas_call(_mlp_body, out_shape=_NP_F32)(
        x, W1, b1.reshape(1, HID), W2, b2.reshape(1, C))

    sdeg = _propagate(ones, zer, row3, col3)

    s, di, v, hid = pl.pallas_call(
        _prep_body, out_shape=(_NP_F32, _NP_F32, _NP_F32, _NP_F32))(
        sdeg[0], sdeg[1], h, temp[0].reshape(1, 1))

    for k in range(K):
        part = _propagate(v, zer, row3, col3)
        v, hid = pl.pallas_call(_acc_body, out_shape=(_NP_F32, _NP_F32))(
            part[0], part[1], di, hid, temp[k + 1].reshape(1, 1))

    return pl.pallas_call(_final_body, out_shape=jax.ShapeDtypeStruct((N, C), jnp.float32))(hid, s)


# speedup vs baseline: 57.0262x; 1.6030x over previous
import jax
import jax.numpy as jnp
from jax import lax
from jax.experimental import pallas as pl
from jax.experimental.pallas import tpu as pltpu
from jax.experimental.pallas import tpu_sc as plsc

N = 10000
D = 128
HID = 128
C = 16
K = 10
E = 320000

NTILES = 32
BLK = 128
NBLK = 80
EPT = NBLK * BLK
E_PAD = NTILES * EPT
NP = 10112
RPT = NP // 16
NBUF = 8
NGRP = NBLK // NBUF

_mesh = plsc.VectorSubcoreMesh(core_axis_name="c", subcore_axis_name="s")
_sc_params = pltpu.CompilerParams(use_tc_tiling_on_sc=False)

_PART = jax.ShapeDtypeStruct((2, NP, C), jnp.float32)
_NP_F32 = jax.ShapeDtypeStruct((NP, C), jnp.float32)


def _edge_phase(row_v, col_v, gbuf, acc, vsh, semg, sems):
    def _gather(j, b):
        pltpu.async_copy(vsh.at[row_v.at[j]], gbuf.at[b], semg.at[b])

    def _gather_wait(j, b):
        pltpu.make_async_copy(vsh.at[row_v.at[j]], gbuf.at[b], semg.at[b]).wait()

    def _scatter(j, b):
        pltpu.async_copy(gbuf.at[b], acc.at[col_v.at[j]], sems.at[b], add=True)

    def _scatter_wait(j, b):
        pltpu.make_async_copy(gbuf.at[b], acc.at[col_v.at[j]], sems.at[b]).wait()

    for b in range(NBUF):
        _gather(b, b)

    @pl.loop(0, NGRP - 1)
    def _(g):
        cur = g * NBUF
        nxt = cur + NBUF
        for b in range(NBUF):
            _gather_wait(cur + b, b)
            _scatter(cur + b, b)
        for b in range(NBUF):
            _scatter_wait(cur + b, b)
            _gather(nxt + b, b)

    last = (NGRP - 1) * NBUF
    for b in range(NBUF):
        _gather_wait(last + b, b)
        _scatter(last + b, b)
    for b in range(NBUF):
        _scatter_wait(last + b, b)


@pl.kernel(
    out_type=_PART,
    mesh=_mesh,
    scratch_types=[
        pltpu.VMEM((NBLK, BLK), jnp.int32),
        pltpu.VMEM((NBLK, BLK), jnp.int32),
        pltpu.VMEM((NBUF, BLK, C), jnp.float32),
        pltpu.VMEM_SHARED((NP, C), jnp.float32),
        pltpu.VMEM_SHARED((NP, C), jnp.float32),
        pltpu.SemaphoreType.DMA((NBUF,)),
        pltpu.SemaphoreType.DMA((NBUF,)),
    ],
    compiler_params=_sc_params,
)
def _propagate(v_hbm, zer_hbm, row_hbm, col_hbm, out_hbm,
               row_v, col_v, gbuf, acc, vsh, semg, sems):
    c = lax.axis_index("c")
    s = lax.axis_index("s")
    tid = s * 2 + c
    pltpu.sync_copy(row_hbm.at[tid], row_v)
    pltpu.sync_copy(col_hbm.at[tid], col_v)

    r0 = s * RPT
    pltpu.sync_copy(v_hbm.at[pl.ds(r0, RPT)], vsh.at[pl.ds(r0, RPT)])

    @pl.when(c == 0)
    def _():
        pltpu.sync_copy(v_hbm.at[pl.ds(r0, RPT)], acc.at[pl.ds(r0, RPT)])

    @pl.when(c != 0)
    def _():
        pltpu.sync_copy(zer_hbm.at[pl.ds(r0, RPT)], acc.at[pl.ds(r0, RPT)])

    plsc.subcore_barrier()
    _edge_phase(row_v, col_v, gbuf, acc, vsh, semg, sems)
    plsc.subcore_barrier()
    pltpu.sync_copy(acc.at[pl.ds(r0, RPT)], out_hbm.at[c, pl.ds(r0, RPT)])


@pl.kernel(
    out_type=_PART,
    mesh=_mesh,
    scratch_types=[
        pltpu.VMEM((NBLK, BLK), jnp.int32),
        pltpu.VMEM((NBLK, BLK), jnp.int32),
        pltpu.VMEM((NBUF, BLK, C), jnp.float32),
        pltpu.VMEM((RPT, C), jnp.float32),
        pltpu.VMEM((RPT, C), jnp.float32),
        pltpu.VMEM((RPT, C), jnp.float32),
        pltpu.VMEM((RPT, C), jnp.float32),
        pltpu.VMEM_SHARED((NP, C), jnp.float32),
        pltpu.VMEM_SHARED((NP, C), jnp.float32),
        pltpu.SemaphoreType.DMA((NBUF,)),
        pltpu.SemaphoreType.DMA((NBUF,)),
        pltpu.SemaphoreType.DMA,
    ],
    compiler_params=_sc_params,
)
def _propagate_fused(p_hbm, di_hbm, zer_hbm, row_hbm, col_hbm, out_hbm,
                     row_v, col_v, gbuf, pa, pb, dv, vbuf, acc, vsh,
                     semg, sems, semi):
    c = lax.axis_index("c")
    s = lax.axis_index("s")
    tid = s * 2 + c
    r0 = s * RPT
    sl = pl.ds(r0, RPT)

    cp_a = pltpu.make_async_copy(p_hbm.at[0, sl], pa, semi)
    cp_b = pltpu.make_async_copy(p_hbm.at[1, sl], pb, semi)
    cp_d = pltpu.make_async_copy(di_hbm.at[sl], dv, semi)
    cp_a.start()
    cp_b.start()
    cp_d.start()
    pltpu.sync_copy(row_hbm.at[tid], row_v)
    pltpu.sync_copy(col_hbm.at[tid], col_v)
    cp_a.wait()
    cp_b.wait()
    cp_d.wait()

    @pl.loop(0, RPT)
    def _(i):
        vbuf[i, :] = (pa[i, :] + pb[i, :]) * dv[i, :]

    pltpu.sync_copy(vbuf, vsh.at[sl])

    @pl.when(c == 0)
    def _():
        pltpu.sync_copy(vbuf, acc.at[sl])

    @pl.when(c != 0)
    def _():
        pltpu.sync_copy(zer_hbm.at[sl], acc.at[sl])

    plsc.subcore_barrier()
    _edge_phase(row_v, col_v, gbuf, acc, vsh, semg, sems)
    plsc.subcore_barrier()
    pltpu.sync_copy(acc.at[sl], out_hbm.at[c, sl])


def _mlp_body(x_ref, w1_ref, b1_ref, w2_ref, b2_ref, o_ref):
    h1 = jnp.dot(x_ref[...], w1_ref[...], preferred_element_type=jnp.float32)
    h1 = jnp.maximum(h1 + b1_ref[...], 0.0)
    o_ref[:N, :] = jnp.dot(h1, w2_ref[...], preferred_element_type=jnp.float32) + b2_ref[...]
    o_ref[N:, :] = jnp.zeros((NP - N, C), jnp.float32)


def _prep_body(s0_ref, s1_ref, h_ref, di_o, v_o):
    deg = s0_ref[...] + s1_ref[...]
    di_o[...] = 1.0 / deg
    v_o[...] = h_ref[...] * lax.rsqrt(deg)


def _accum_body(s0_ref, s1_ref, h_ref, t_ref, *rest):
    part_refs = rest[:-1]
    o_ref = rest[-1]
    deg = s0_ref[...] + s1_ref[...]
    sc = lax.rsqrt(deg)
    hid = t_ref[0, 0] * deg * sc * h_ref[...]
    for k in range(K):
        pk = part_refs[k]
        hid = hid + t_ref[0, k + 1] * (pk[0] + pk[1])
    o_ref[...] = hid * sc


def _softmax_body(hd_ref, o_ref):
    hd = hd_ref[:N, :]
    m = jnp.max(hd, axis=1, keepdims=True)
    lse = jnp.log(jnp.sum(jnp.exp(hd - m), axis=1, keepdims=True))
    o_ref[...] = hd - m - lse


def kernel(x, edge_index, W1, b1, W2, b2, temp):
    row = edge_index[0]
    col = edge_index[1]
    pad = E_PAD - E
    rowp = jnp.concatenate([row, jnp.zeros((pad,), row.dtype)])
    colp = jnp.concatenate([col, jnp.full((pad,), N, col.dtype)])
    row3 = rowp.reshape(NTILES, NBLK, BLK)
    col3 = colp.reshape(NTILES, NBLK, BLK)

    zer = jnp.zeros((NP, C), jnp.float32)
    ones = jnp.ones((NP, C), jnp.float32)

    h = pl.pallas_call(_mlp_body, out_shape=_NP_F32)(
        x, W1, b1.reshape(1, HID), W2, b2.reshape(1, C))

    sdeg = _propagate(ones, zer, row3, col3)

    di, v0 = pl.pallas_call(_prep_body, out_shape=(_NP_F32, _NP_F32))(
        sdeg[0], sdeg[1], h)

    parts = [_propagate(v0, zer, row3, col3)]
    for _ in range(K - 1):
        parts.append(_propagate_fused(parts[-1], di, zer, row3, col3))

    flat = (NP * C // 128, 128)
    hd = pl.pallas_call(
        _accum_body, out_shape=jax.ShapeDtypeStruct(flat, jnp.float32))(
        sdeg[0].reshape(flat), sdeg[1].reshape(flat), h.reshape(flat),
        temp.reshape(1, K + 1), *[p.reshape((2,) + flat) for p in parts])

    return pl.pallas_call(
        _softmax_body, out_shape=jax.ShapeDtypeStruct((N, C), jnp.float32))(
        hd.reshape(NP, C))


# speedup vs baseline: 57.7910x; 1.0134x over previous
import jax
import jax.numpy as jnp
from jax import lax
from jax.experimental import pallas as pl
from jax.experimental.pallas import tpu as pltpu
from jax.experimental.pallas import tpu_sc as plsc

N = 10000
D = 128
HID = 128
C = 16
K = 10
E = 320000

NTILES = 32
BLK = 128
NBLK = 80
EPT = NBLK * BLK
E_PAD = NTILES * EPT
NP = 10112
RPT = NP // 16
NBUF = 8
NGRP = NBLK // NBUF

_mesh = plsc.VectorSubcoreMesh(core_axis_name="c", subcore_axis_name="s")
_sc_params = pltpu.CompilerParams(use_tc_tiling_on_sc=False)

_PART = jax.ShapeDtypeStruct((2, NP, C), jnp.float32)
_NP_F32 = jax.ShapeDtypeStruct((NP, C), jnp.float32)


def _edge_phase(row_v, col_v, gbuf, acc, vsh, semg, sems):
    def _gather(j, b):
        pltpu.async_copy(vsh.at[row_v.at[j]], gbuf.at[b], semg.at[b])

    def _gather_wait(j, b):
        pltpu.make_async_copy(vsh.at[row_v.at[j]], gbuf.at[b], semg.at[b]).wait()

    def _scatter(j, b):
        pltpu.async_copy(gbuf.at[b], acc.at[col_v.at[j]], sems.at[b], add=True)

    def _scatter_wait(j, b):
        pltpu.make_async_copy(gbuf.at[b], acc.at[col_v.at[j]], sems.at[b]).wait()

    for b in range(NBUF):
        _gather(b, b)

    @pl.loop(0, NGRP - 1)
    def _(g):
        cur = g * NBUF
        nxt = cur + NBUF
        for b in range(NBUF):
            _gather_wait(cur + b, b)
            _scatter(cur + b, b)
        for b in range(NBUF):
            _scatter_wait(cur + b, b)
            _gather(nxt + b, b)

    last = (NGRP - 1) * NBUF
    for b in range(NBUF):
        _gather_wait(last + b, b)
        _scatter(last + b, b)
    for b in range(NBUF):
        _scatter_wait(last + b, b)


@pl.kernel(
    out_type=_PART,
    mesh=_mesh,
    scratch_types=[
        pltpu.VMEM((NBLK, BLK), jnp.int32),
        pltpu.VMEM((NBLK, BLK), jnp.int32),
        pltpu.VMEM((NBUF, BLK, C), jnp.float32),
        pltpu.VMEM_SHARED((NP, C), jnp.float32),
        pltpu.VMEM_SHARED((NP, C), jnp.float32),
        pltpu.SemaphoreType.DMA((NBUF,)),
        pltpu.SemaphoreType.DMA((NBUF,)),
    ],
    compiler_params=_sc_params,
)
def _propagate(v_hbm, zer_hbm, row_hbm, col_hbm, out_hbm,
               row_v, col_v, gbuf, acc, vsh, semg, sems):
    c = lax.axis_index("c")
    s = lax.axis_index("s")
    tid = s * 2 + c
    pltpu.sync_copy(row_hbm.at[tid], row_v)
    pltpu.sync_copy(col_hbm.at[tid], col_v)

    r0 = s * RPT
    pltpu.sync_copy(v_hbm.at[pl.ds(r0, RPT)], vsh.at[pl.ds(r0, RPT)])

    @pl.when(c == 0)
    def _():
        pltpu.sync_copy(v_hbm.at[pl.ds(r0, RPT)], acc.at[pl.ds(r0, RPT)])

    @pl.when(c != 0)
    def _():
        pltpu.sync_copy(zer_hbm.at[pl.ds(r0, RPT)], acc.at[pl.ds(r0, RPT)])

    plsc.subcore_barrier()
    _edge_phase(row_v, col_v, gbuf, acc, vsh, semg, sems)
    plsc.subcore_barrier()
    pltpu.sync_copy(acc.at[pl.ds(r0, RPT)], out_hbm.at[c, pl.ds(r0, RPT)])


@pl.kernel(
    out_type=_PART,
    mesh=_mesh,
    scratch_types=[
        pltpu.VMEM((NBLK, BLK), jnp.int32),
        pltpu.VMEM((NBLK, BLK), jnp.int32),
        pltpu.VMEM((NBUF, BLK, C), jnp.float32),
        pltpu.VMEM((RPT, C), jnp.float32),
        pltpu.VMEM((RPT, C), jnp.float32),
        pltpu.VMEM((RPT, C), jnp.float32),
        pltpu.VMEM((RPT, C), jnp.float32),
        pltpu.VMEM_SHARED((NP, C), jnp.float32),
        pltpu.VMEM_SHARED((NP, C), jnp.float32),
        pltpu.SemaphoreType.DMA((NBUF,)),
        pltpu.SemaphoreType.DMA((NBUF,)),
        pltpu.SemaphoreType.DMA,
    ],
    compiler_params=_sc_params,
)
def _propagate_fused(p_hbm, di_hbm, zer_hbm, row_hbm, col_hbm, out_hbm,
                     row_v, col_v, gbuf, pa, pb, dv, vbuf, acc, vsh,
                     semg, sems, semi):
    c = lax.axis_index("c")
    s = lax.axis_index("s")
    tid = s * 2 + c
    r0 = s * RPT
    sl = pl.ds(r0, RPT)

    cp_a = pltpu.make_async_copy(p_hbm.at[0, sl], pa, semi)
    cp_b = pltpu.make_async_copy(p_hbm.at[1, sl], pb, semi)
    cp_d = pltpu.make_async_copy(di_hbm.at[sl], dv, semi)
    cp_a.start()
    cp_b.start()
    cp_d.start()
    pltpu.sync_copy(row_hbm.at[tid], row_v)
    pltpu.sync_copy(col_hbm.at[tid], col_v)
    cp_a.wait()
    cp_b.wait()
    cp_d.wait()

    @pl.loop(0, RPT)
    def _(i):
        vbuf[i, :] = (pa[i, :] + pb[i, :]) * dv[i, :]

    pltpu.sync_copy(vbuf, vsh.at[sl])

    @pl.when(c == 0)
    def _():
        pltpu.sync_copy(vbuf, acc.at[sl])

    @pl.when(c != 0)
    def _():
        pltpu.sync_copy(zer_hbm.at[sl], acc.at[sl])

    plsc.subcore_barrier()
    _edge_phase(row_v, col_v, gbuf, acc, vsh, semg, sems)
    plsc.subcore_barrier()
    pltpu.sync_copy(acc.at[sl], out_hbm.at[c, sl])


@pl.kernel(
    out_type=_PART,
    mesh=_mesh,
    scratch_types=[
        pltpu.VMEM((NBLK, BLK), jnp.int32),
        pltpu.VMEM((BLK, C), jnp.float32),
        pltpu.VMEM_SHARED((NP, C), jnp.float32),
        pltpu.SemaphoreType.DMA((NBUF,)),
    ],
    compiler_params=_sc_params,
)
def _degree(ones_hbm, zer_hbm, col_hbm, out_hbm, col_v, obuf, acc, sems):
    c = lax.axis_index("c")
    s = lax.axis_index("s")
    tid = s * 2 + c
    r0 = s * RPT
    sl = pl.ds(r0, RPT)
    pltpu.sync_copy(col_hbm.at[tid], col_v)
    pltpu.sync_copy(ones_hbm.at[pl.ds(0, BLK)], obuf)

    @pl.when(c == 0)
    def _():
        pltpu.sync_copy(ones_hbm.at[sl], acc.at[sl])

    @pl.when(c != 0)
    def _():
        pltpu.sync_copy(zer_hbm.at[sl], acc.at[sl])

    plsc.subcore_barrier()

    def _scatter(j, b):
        pltpu.async_copy(obuf, acc.at[col_v.at[j]], sems.at[b], add=True)

    def _scatter_wait(j, b):
        pltpu.make_async_copy(obuf, acc.at[col_v.at[j]], sems.at[b]).wait()

    for b in range(NBUF):
        _scatter(b, b)

    @pl.loop(0, NGRP - 1)
    def _(g):
        cur = g * NBUF
        nxt = cur + NBUF
        for b in range(NBUF):
            _scatter_wait(cur + b, b)
            _scatter(nxt + b, b)

    last = (NGRP - 1) * NBUF
    for b in range(NBUF):
        _scatter_wait(last + b, b)

    plsc.subcore_barrier()
    pltpu.sync_copy(acc.at[sl], out_hbm.at[c, sl])


def _mlp_body(x_ref, w1_ref, b1_ref, w2_ref, b2_ref, o_ref):
    h1 = jnp.dot(x_ref[...], w1_ref[...], preferred_element_type=jnp.float32)
    h1 = jnp.maximum(h1 + b1_ref[...], 0.0)
    o_ref[:N, :] = jnp.dot(h1, w2_ref[...], preferred_element_type=jnp.float32) + b2_ref[...]
    o_ref[N:, :] = jnp.zeros((NP - N, C), jnp.float32)


def _prep_body(s0_ref, s1_ref, h_ref, di_o, v_o):
    deg = s0_ref[...] + s1_ref[...]
    di_o[...] = 1.0 / deg
    v_o[...] = h_ref[...] * lax.rsqrt(deg)


def _accum_body(s0_ref, s1_ref, h_ref, t_ref, *rest):
    part_refs = rest[:-1]
    o_ref = rest[-1]
    deg = s0_ref[...] + s1_ref[...]
    sc = lax.rsqrt(deg)
    hid = t_ref[0, 0] * deg * sc * h_ref[...]
    for k in range(K):
        pk = part_refs[k]
        hid = hid + t_ref[0, k + 1] * (pk[0] + pk[1])
    o_ref[...] = hid * sc


def _softmax_body(hd_ref, o_ref):
    hd = hd_ref[:N, :]
    m = jnp.max(hd, axis=1, keepdims=True)
    lse = jnp.log(jnp.sum(jnp.exp(hd - m), axis=1, keepdims=True))
    o_ref[...] = hd - m - lse


def kernel(x, edge_index, W1, b1, W2, b2, temp):
    row = edge_index[0]
    col = edge_index[1]
    pad = E_PAD - E
    rowp = jnp.concatenate([row, jnp.zeros((pad,), row.dtype)])
    colp = jnp.concatenate([col, jnp.full((pad,), N, col.dtype)])
    row3 = rowp.reshape(NTILES, NBLK, BLK)
    col3 = colp.reshape(NTILES, NBLK, BLK)

    zer = jnp.zeros((NP, C), jnp.float32)
    ones = jnp.ones((NP, C), jnp.float32)

    h = pl.pallas_call(_mlp_body, out_shape=_NP_F32)(
        x, W1, b1.reshape(1, HID), W2, b2.reshape(1, C))

    sdeg = _degree(ones, zer, col3)

    di, v0 = pl.pallas_call(_prep_body, out_shape=(_NP_F32, _NP_F32))(
        sdeg[0], sdeg[1], h)

    parts = [_propagate(v0, zer, row3, col3)]
    for _ in range(K - 1):
        parts.append(_propagate_fused(parts[-1], di, zer, row3, col3))

    flat = (NP * C // 128, 128)
    hd = pl.pallas_call(
        _accum_body, out_shape=jax.ShapeDtypeStruct(flat, jnp.float32))(
        sdeg[0].reshape(flat), sdeg[1].reshape(flat), h.reshape(flat),
        temp.reshape(1, K + 1), *[p.reshape((2,) + flat) for p in parts])

    return pl.pallas_call(
        _softmax_body, out_shape=jax.ShapeDtypeStruct((N, C), jnp.float32))(
        hd.reshape(NP, C))


# speedup vs baseline: 59.5555x; 1.0305x over previous
import jax
import jax.numpy as jnp
from jax import lax
from jax.experimental import pallas as pl
from jax.experimental.pallas import tpu as pltpu
from jax.experimental.pallas import tpu_sc as plsc

N = 10000
D = 128
HID = 128
C = 16
K = 10
E = 320000

NTILES = 32
BLK = 128
NBLK = 80
EPT = NBLK * BLK
E_PAD = NTILES * EPT
NP = 10112
RPT = NP // 16
NBUF = 8
NGRP = NBLK // NBUF
EPC = E // NTILES

_mesh = plsc.VectorSubcoreMesh(core_axis_name="c", subcore_axis_name="s")
_sc_params = pltpu.CompilerParams(use_tc_tiling_on_sc=False)

_PART = jax.ShapeDtypeStruct((2, NP, C), jnp.float32)
_NP_F32 = jax.ShapeDtypeStruct((NP, C), jnp.float32)


def _edge_phase(row_v, col_v, gbuf, acc, vsh, semg, sems):
    def _gather(j, b):
        pltpu.async_copy(vsh.at[row_v.at[j]], gbuf.at[b], semg.at[b])

    def _gather_wait(j, b):
        pltpu.make_async_copy(vsh.at[row_v.at[j]], gbuf.at[b], semg.at[b]).wait()

    def _scatter(j, b):
        pltpu.async_copy(gbuf.at[b], acc.at[col_v.at[j]], sems.at[b], add=True)

    def _scatter_wait(j, b):
        pltpu.make_async_copy(gbuf.at[b], acc.at[col_v.at[j]], sems.at[b]).wait()

    for b in range(NBUF):
        _gather(b, b)

    @pl.loop(0, NGRP - 1)
    def _(g):
        cur = g * NBUF
        nxt = cur + NBUF
        for b in range(NBUF):
            _gather_wait(cur + b, b)
            _scatter(cur + b, b)
        for b in range(NBUF):
            _scatter_wait(cur + b, b)
            _gather(nxt + b, b)

    last = (NGRP - 1) * NBUF
    for b in range(NBUF):
        _gather_wait(last + b, b)
        _scatter(last + b, b)
    for b in range(NBUF):
        _scatter_wait(last + b, b)


@pl.kernel(
    out_type=_PART,
    mesh=_mesh,
    scratch_types=[
        pltpu.VMEM((NBLK, BLK), jnp.int32),
        pltpu.VMEM((NBLK, BLK), jnp.int32),
        pltpu.VMEM((NBUF, BLK, C), jnp.float32),
        pltpu.VMEM_SHARED((NP, C), jnp.float32),
        pltpu.VMEM_SHARED((NP, C), jnp.float32),
        pltpu.SemaphoreType.DMA((NBUF,)),
        pltpu.SemaphoreType.DMA((NBUF,)),
    ],
    compiler_params=_sc_params,
)
def _propagate(v_hbm, zer_hbm, row_hbm, col_hbm, out_hbm,
               row_v, col_v, gbuf, acc, vsh, semg, sems):
    c = lax.axis_index("c")
    s = lax.axis_index("s")
    tid = s * 2 + c
    pltpu.sync_copy(row_hbm.at[tid], row_v)
    pltpu.sync_copy(col_hbm.at[tid], col_v)

    r0 = s * RPT
    pltpu.sync_copy(v_hbm.at[pl.ds(r0, RPT)], vsh.at[pl.ds(r0, RPT)])

    @pl.when(c == 0)
    def _():
        pltpu.sync_copy(v_hbm.at[pl.ds(r0, RPT)], acc.at[pl.ds(r0, RPT)])

    @pl.when(c != 0)
    def _():
        pltpu.sync_copy(zer_hbm.at[pl.ds(r0, RPT)], acc.at[pl.ds(r0, RPT)])

    plsc.subcore_barrier()
    _edge_phase(row_v, col_v, gbuf, acc, vsh, semg, sems)
    plsc.subcore_barrier()
    pltpu.sync_copy(acc.at[pl.ds(r0, RPT)], out_hbm.at[c, pl.ds(r0, RPT)])


@pl.kernel(
    out_type=_PART,
    mesh=_mesh,
    scratch_types=[
        pltpu.VMEM((NBLK, BLK), jnp.int32),
        pltpu.VMEM((NBLK, BLK), jnp.int32),
        pltpu.VMEM((NBUF, BLK, C), jnp.float32),
        pltpu.VMEM((RPT, C), jnp.float32),
        pltpu.VMEM((RPT, C), jnp.float32),
        pltpu.VMEM((RPT, C), jnp.float32),
        pltpu.VMEM((RPT, C), jnp.float32),
        pltpu.VMEM_SHARED((NP, C), jnp.float32),
        pltpu.VMEM_SHARED((NP, C), jnp.float32),
        pltpu.SemaphoreType.DMA((NBUF,)),
        pltpu.SemaphoreType.DMA((NBUF,)),
        pltpu.SemaphoreType.DMA,
    ],
    compiler_params=_sc_params,
)
def _propagate_fused(p_hbm, di_hbm, zer_hbm, row_hbm, col_hbm, out_hbm,
                     row_v, col_v, gbuf, pa, pb, dv, vbuf, acc, vsh,
                     semg, sems, semi):
    c = lax.axis_index("c")
    s = lax.axis_index("s")
    tid = s * 2 + c
    r0 = s * RPT
    sl = pl.ds(r0, RPT)

    cp_a = pltpu.make_async_copy(p_hbm.at[0, sl], pa, semi)
    cp_b = pltpu.make_async_copy(p_hbm.at[1, sl], pb, semi)
    cp_d = pltpu.make_async_copy(di_hbm.at[sl], dv, semi)
    cp_a.start()
    cp_b.start()
    cp_d.start()
    pltpu.sync_copy(row_hbm.at[tid], row_v)
    pltpu.sync_copy(col_hbm.at[tid], col_v)
    cp_a.wait()
    cp_b.wait()
    cp_d.wait()

    @pl.loop(0, RPT)
    def _(i):
        vbuf[i, :] = (pa[i, :] + pb[i, :]) * dv[i, :]

    pltpu.sync_copy(vbuf, vsh.at[sl])

    @pl.when(c == 0)
    def _():
        pltpu.sync_copy(vbuf, acc.at[sl])

    @pl.when(c != 0)
    def _():
        pltpu.sync_copy(zer_hbm.at[sl], acc.at[sl])

    plsc.subcore_barrier()
    _edge_phase(row_v, col_v, gbuf, acc, vsh, semg, sems)
    plsc.subcore_barrier()
    pltpu.sync_copy(acc.at[sl], out_hbm.at[c, sl])


@pl.kernel(
    out_type=_PART,
    mesh=_mesh,
    scratch_types=[
        pltpu.VMEM((NBLK, BLK), jnp.int32),
        pltpu.VMEM((BLK, C), jnp.float32),
        pltpu.VMEM_SHARED((NP, C), jnp.float32),
        pltpu.SemaphoreType.DMA((NBUF,)),
    ],
    compiler_params=_sc_params,
)
def _degree(ones_hbm, zer_hbm, col_hbm, out_hbm, col_v, obuf, acc, sems):
    c = lax.axis_index("c")
    s = lax.axis_index("s")
    tid = s * 2 + c
    r0 = s * RPT
    sl = pl.ds(r0, RPT)
    pltpu.sync_copy(col_hbm.at[tid], col_v)
    pltpu.sync_copy(ones_hbm.at[pl.ds(0, BLK)], obuf)

    @pl.when(c == 0)
    def _():
        pltpu.sync_copy(ones_hbm.at[sl], acc.at[sl])

    @pl.when(c != 0)
    def _():
        pltpu.sync_copy(zer_hbm.at[sl], acc.at[sl])

    plsc.subcore_barrier()

    def _scatter(j, b):
        pltpu.async_copy(obuf, acc.at[col_v.at[j]], sems.at[b], add=True)

    def _scatter_wait(j, b):
        pltpu.make_async_copy(obuf, acc.at[col_v.at[j]], sems.at[b]).wait()

    for b in range(NBUF):
        _scatter(b, b)

    @pl.loop(0, NGRP - 1)
    def _(g):
        cur = g * NBUF
        nxt = cur + NBUF
        for b in range(NBUF):
            _scatter_wait(cur + b, b)
            _scatter(nxt + b, b)

    last = (NGRP - 1) * NBUF
    for b in range(NBUF):
        _scatter_wait(last + b, b)

    plsc.subcore_barrier()
    pltpu.sync_copy(acc.at[sl], out_hbm.at[c, sl])


def _edges_body(ei_ref, row_o, col_o):
    row_o[:, :EPC] = ei_ref[0]
    row_o[:, EPC:] = jnp.zeros((NTILES, EPT - EPC), jnp.int32)
    col_o[:, :EPC] = ei_ref[1]
    col_o[:, EPC:] = jnp.full((NTILES, EPT - EPC), N, jnp.int32)


def _mlp_body(x_ref, w1_ref, b1_ref, w2_ref, b2_ref, o_ref):
    h1 = jnp.dot(x_ref[...], w1_ref[...], preferred_element_type=jnp.float32)
    h1 = jnp.maximum(h1 + b1_ref[...], 0.0)
    o_ref[:N, :] = jnp.dot(h1, w2_ref[...], preferred_element_type=jnp.float32) + b2_ref[...]
    o_ref[N:, :] = jnp.zeros((NP - N, C), jnp.float32)


def _prep_body(s0_ref, s1_ref, h_ref, di_o, v_o):
    deg = s0_ref[...] + s1_ref[...]
    di_o[...] = 1.0 / deg
    v_o[...] = h_ref[...] * lax.rsqrt(deg)


def _accum_body(s0_ref, s1_ref, h_ref, t_ref, *rest):
    part_refs = rest[:-1]
    o_ref = rest[-1]
    deg = s0_ref[...] + s1_ref[...]
    sc = lax.rsqrt(deg)
    hid = t_ref[0, 0] * deg * sc * h_ref[...]
    for k in range(K):
        pk = part_refs[k]
        hid = hid + t_ref[0, k + 1] * (pk[0] + pk[1])
    o_ref[...] = hid * sc


def _softmax_body(hd_ref, o_ref):
    hd = hd_ref[:N, :]
    m = jnp.max(hd, axis=1, keepdims=True)
    lse = jnp.log(jnp.sum(jnp.exp(hd - m), axis=1, keepdims=True))
    o_ref[...] = hd - m - lse


def kernel(x, edge_index, W1, b1, W2, b2, temp):
    row2, col2 = pl.pallas_call(
        _edges_body,
        out_shape=(jax.ShapeDtypeStruct((NTILES, EPT), jnp.int32),
                   jax.ShapeDtypeStruct((NTILES, EPT), jnp.int32)))(
        edge_index.reshape(2, NTILES, EPC))
    row3 = row2.reshape(NTILES, NBLK, BLK)
    col3 = col2.reshape(NTILES, NBLK, BLK)

    zer = jnp.zeros((NP, C), jnp.float32)
    ones = jnp.ones((NP, C), jnp.float32)

    h = pl.pallas_call(_mlp_body, out_shape=_NP_F32)(
        x, W1, b1.reshape(1, HID), W2, b2.reshape(1, C))

    sdeg = _degree(ones, zer, col3)

    di, v0 = pl.pallas_call(_prep_body, out_shape=(_NP_F32, _NP_F32))(
        sdeg[0], sdeg[1], h)

    parts = [_propagate(v0, zer, row3, col3)]
    for _ in range(K - 1):
        parts.append(_propagate_fused(parts[-1], di, zer, row3, col3))

    flat = (NP * C // 128, 128)
    hd = pl.pallas_call(
        _accum_body, out_shape=jax.ShapeDtypeStruct(flat, jnp.float32))(
        sdeg[0].reshape(flat), sdeg[1].reshape(flat), h.reshape(flat),
        temp.reshape(1, K + 1), *[p.reshape((2,) + flat) for p in parts])

    return pl.pallas_call(
        _softmax_body, out_shape=jax.ShapeDtypeStruct((N, C), jnp.float32))(
        hd.reshape(NP, C))


# speedup vs baseline: 63.9728x; 1.0742x over previous
import jax
import jax.numpy as jnp
from jax import lax
from jax.experimental import pallas as pl
from jax.experimental.pallas import tpu as pltpu
from jax.experimental.pallas import tpu_sc as plsc

N = 10000
D = 128
HID = 128
C = 16
K = 10
E = 320000

NTILES = 32
BLK = 128
NBLK = 80
EPT = NBLK * BLK
E_PAD = NTILES * EPT
NP = 10112
RPT = NP // 16
NBUF = 8
NGRP = NBLK // NBUF
EPC = E // NTILES

_mesh = plsc.VectorSubcoreMesh(core_axis_name="c", subcore_axis_name="s")
_sc_params = pltpu.CompilerParams(use_tc_tiling_on_sc=False)

_PART = jax.ShapeDtypeStruct((2, NP, C), jnp.float32)
_NP_F32 = jax.ShapeDtypeStruct((NP, C), jnp.float32)


def _edge_phase(row_v, col_v, gbuf, acc, vsh, semg, sems):
    def _gather(j, b):
        pltpu.async_copy(vsh.at[row_v.at[j]], gbuf.at[b], semg.at[b])

    def _gather_wait(j, b):
        pltpu.make_async_copy(vsh.at[row_v.at[j]], gbuf.at[b], semg.at[b]).wait()

    def _scatter(j, b):
        pltpu.async_copy(gbuf.at[b], acc.at[col_v.at[j]], sems.at[b], add=True)

    def _scatter_wait(j, b):
        pltpu.make_async_copy(gbuf.at[b], acc.at[col_v.at[j]], sems.at[b]).wait()

    for b in range(NBUF):
        _gather(b, b)

    @pl.loop(0, NGRP - 1)
    def _(g):
        cur = g * NBUF
        nxt = cur + NBUF
        for b in range(NBUF):
            _gather_wait(cur + b, b)
            _scatter(cur + b, b)
        for b in range(NBUF):
            _scatter_wait(cur + b, b)
            _gather(nxt + b, b)

    last = (NGRP - 1) * NBUF
    for b in range(NBUF):
        _gather_wait(last + b, b)
        _scatter(last + b, b)
    for b in range(NBUF):
        _scatter_wait(last + b, b)


@pl.kernel(
    out_type=(_PART, _NP_F32),
    mesh=_mesh,
    scratch_types=[
        pltpu.VMEM((NBLK, BLK), jnp.int32),
        pltpu.VMEM((NBLK, BLK), jnp.int32),
        pltpu.VMEM((NBUF, BLK, C), jnp.float32),
        pltpu.VMEM((RPT, C), jnp.float32),
        pltpu.VMEM((RPT, C), jnp.float32),
        pltpu.VMEM((RPT, C), jnp.float32),
        pltpu.VMEM((RPT, C), jnp.float32),
        pltpu.VMEM((RPT, C), jnp.float32),
        pltpu.VMEM_SHARED((NP, C), jnp.float32),
        pltpu.VMEM_SHARED((NP, C), jnp.float32),
        pltpu.SemaphoreType.DMA((NBUF,)),
        pltpu.SemaphoreType.DMA((NBUF,)),
        pltpu.SemaphoreType.DMA,
    ],
    compiler_params=_sc_params,
)
def _round0(degp_hbm, h_hbm, zer_hbm, row_hbm, col_hbm, out_hbm, di_hbm,
            row_v, col_v, gbuf, pa, pb, hv, vbuf, dibuf, acc, vsh,
            semg, sems, semi):
    c = lax.axis_index("c")
    s = lax.axis_index("s")
    tid = s * 2 + c
    r0 = s * RPT
    sl = pl.ds(r0, RPT)

    cp_a = pltpu.make_async_copy(degp_hbm.at[0, sl], pa, semi)
    cp_b = pltpu.make_async_copy(degp_hbm.at[1, sl], pb, semi)
    cp_h = pltpu.make_async_copy(h_hbm.at[sl], hv, semi)
    cp_a.start()
    cp_b.start()
    cp_h.start()
    pltpu.sync_copy(row_hbm.at[tid], row_v)
    pltpu.sync_copy(col_hbm.at[tid], col_v)
    cp_a.wait()
    cp_b.wait()
    cp_h.wait()

    @pl.loop(0, RPT)
    def _(i):
        d = pa[i, :] + pb[i, :]
        bits = lax.bitcast_convert_type(d, jnp.int32)
        y = lax.bitcast_convert_type(
            jnp.int32(0x5F3759DF) - lax.shift_right_logical(bits, 1),
            jnp.float32)
        y = y * (1.5 - 0.5 * d * y * y)
        y = y * (1.5 - 0.5 * d * y * y)
        y = y * (1.5 - 0.5 * d * y * y)
        vbuf[i, :] = hv[i, :] * y
        dibuf[i, :] = y * y

    pltpu.sync_copy(vbuf, vsh.at[sl])

    @pl.when(c == 0)
    def _():
        pltpu.sync_copy(vbuf, acc.at[sl])
        pltpu.sync_copy(dibuf, di_hbm.at[sl])

    @pl.when(c != 0)
    def _():
        pltpu.sync_copy(zer_hbm.at[sl], acc.at[sl])

    plsc.subcore_barrier()
    _edge_phase(row_v, col_v, gbuf, acc, vsh, semg, sems)
    plsc.subcore_barrier()
    pltpu.sync_copy(acc.at[sl], out_hbm.at[c, sl])


@pl.kernel(
    out_type=_PART,
    mesh=_mesh,
    scratch_types=[
        pltpu.VMEM((NBLK, BLK), jnp.int32),
        pltpu.VMEM((NBLK, BLK), jnp.int32),
        pltpu.VMEM((NBUF, BLK, C), jnp.float32),
        pltpu.VMEM((RPT, C), jnp.float32),
        pltpu.VMEM((RPT, C), jnp.float32),
        pltpu.VMEM((RPT, C), jnp.float32),
        pltpu.VMEM((RPT, C), jnp.float32),
        pltpu.VMEM_SHARED((NP, C), jnp.float32),
        pltpu.VMEM_SHARED((NP, C), jnp.float32),
        pltpu.SemaphoreType.DMA((NBUF,)),
        pltpu.SemaphoreType.DMA((NBUF,)),
        pltpu.SemaphoreType.DMA,
    ],
    compiler_params=_sc_params,
)
def _propagate_fused(p_hbm, di_hbm, zer_hbm, row_hbm, col_hbm, out_hbm,
                     row_v, col_v, gbuf, pa, pb, dv, vbuf, acc, vsh,
                     semg, sems, semi):
    c = lax.axis_index("c")
    s = lax.axis_index("s")
    tid = s * 2 + c
    r0 = s * RPT
    sl = pl.ds(r0, RPT)

    cp_a = pltpu.make_async_copy(p_hbm.at[0, sl], pa, semi)
    cp_b = pltpu.make_async_copy(p_hbm.at[1, sl], pb, semi)
    cp_d = pltpu.make_async_copy(di_hbm.at[sl], dv, semi)
    cp_a.start()
    cp_b.start()
    cp_d.start()
    pltpu.sync_copy(row_hbm.at[tid], row_v)
    pltpu.sync_copy(col_hbm.at[tid], col_v)
    cp_a.wait()
    cp_b.wait()
    cp_d.wait()

    @pl.loop(0, RPT)
    def _(i):
        vbuf[i, :] = (pa[i, :] + pb[i, :]) * dv[i, :]

    pltpu.sync_copy(vbuf, vsh.at[sl])

    @pl.when(c == 0)
    def _():
        pltpu.sync_copy(vbuf, acc.at[sl])

    @pl.when(c != 0)
    def _():
        pltpu.sync_copy(zer_hbm.at[sl], acc.at[sl])

    plsc.subcore_barrier()
    _edge_phase(row_v, col_v, gbuf, acc, vsh, semg, sems)
    plsc.subcore_barrier()
    pltpu.sync_copy(acc.at[sl], out_hbm.at[c, sl])


@pl.kernel(
    out_type=_PART,
    mesh=_mesh,
    scratch_types=[
        pltpu.VMEM((NBLK, BLK), jnp.int32),
        pltpu.VMEM((BLK, C), jnp.float32),
        pltpu.VMEM_SHARED((NP, C), jnp.float32),
        pltpu.SemaphoreType.DMA((NBUF,)),
    ],
    compiler_params=_sc_params,
)
def _degree(ones_hbm, zer_hbm, col_hbm, out_hbm, col_v, obuf, acc, sems):
    c = lax.axis_index("c")
    s = lax.axis_index("s")
    tid = s * 2 + c
    r0 = s * RPT
    sl = pl.ds(r0, RPT)
    pltpu.sync_copy(col_hbm.at[tid], col_v)
    pltpu.sync_copy(ones_hbm.at[pl.ds(0, BLK)], obuf)

    @pl.when(c == 0)
    def _():
        pltpu.sync_copy(ones_hbm.at[sl], acc.at[sl])

    @pl.when(c != 0)
    def _():
        pltpu.sync_copy(zer_hbm.at[sl], acc.at[sl])

    plsc.subcore_barrier()

    def _scatter(j, b):
        pltpu.async_copy(obuf, acc.at[col_v.at[j]], sems.at[b], add=True)

    def _scatter_wait(j, b):
        pltpu.make_async_copy(obuf, acc.at[col_v.at[j]], sems.at[b]).wait()

    for b in range(NBUF):
        _scatter(b, b)

    @pl.loop(0, NGRP - 1)
    def _(g):
        cur = g * NBUF
        nxt = cur + NBUF
        for b in range(NBUF):
            _scatter_wait(cur + b, b)
            _scatter(nxt + b, b)

    last = (NGRP - 1) * NBUF
    for b in range(NBUF):
        _scatter_wait(last + b, b)

    plsc.subcore_barrier()
    pltpu.sync_copy(acc.at[sl], out_hbm.at[c, sl])


def _edges_body(ei_ref, row_o, col_o):
    row_o[:, :EPC] = ei_ref[0]
    row_o[:, EPC:] = jnp.zeros((NTILES, EPT - EPC), jnp.int32)
    col_o[:, :EPC] = ei_ref[1]
    col_o[:, EPC:] = jnp.full((NTILES, EPT - EPC), N, jnp.int32)


def _mlp_body(x_ref, w1_ref, b1_ref, w2_ref, b2_ref, o_ref):
    h1 = jnp.dot(x_ref[...], w1_ref[...], preferred_element_type=jnp.float32)
    h1 = jnp.maximum(h1 + b1_ref[...], 0.0)
    o_ref[:N, :] = jnp.dot(h1, w2_ref[...], preferred_element_type=jnp.float32) + b2_ref[...]
    o_ref[N:, :] = jnp.zeros((NP - N, C), jnp.float32)


def _accum_body(s0_ref, s1_ref, h_ref, t_ref, *rest):
    part_refs = rest[:-1]
    o_ref = rest[-1]
    deg = s0_ref[...] + s1_ref[...]
    sc = lax.rsqrt(deg)
    hid = t_ref[0, 0] * deg * sc * h_ref[...]
    for k in range(K):
        pk = part_refs[k]
        hid = hid + t_ref[0, k + 1] * (pk[0] + pk[1])
    o_ref[...] = hid * sc


def _softmax_body(hd_ref, o_ref):
    hd = hd_ref[:N, :]
    m = jnp.max(hd, axis=1, keepdims=True)
    lse = jnp.log(jnp.sum(jnp.exp(hd - m), axis=1, keepdims=True))
    o_ref[...] = hd - m - lse


def kernel(x, edge_index, W1, b1, W2, b2, temp):
    row2, col2 = pl.pallas_call(
        _edges_body,
        out_shape=(jax.ShapeDtypeStruct((NTILES, EPT), jnp.int32),
                   jax.ShapeDtypeStruct((NTILES, EPT), jnp.int32)))(
        edge_index.reshape(2, NTILES, EPC))
    row3 = row2.reshape(NTILES, NBLK, BLK)
    col3 = col2.reshape(NTILES, NBLK, BLK)

    zer = jnp.zeros((NP, C), jnp.float32)
    ones = jnp.ones((NP, C), jnp.float32)

    h = pl.pallas_call(_mlp_body, out_shape=_NP_F32)(
        x, W1, b1.reshape(1, HID), W2, b2.reshape(1, C))

    sdeg = _degree(ones, zer, col3)

    parts0, di = _round0(sdeg, h, zer, row3, col3)
    parts = [parts0]
    for _ in range(K - 1):
        parts.append(_propagate_fused(parts[-1], di, zer, row3, col3))

    flat = (NP * C // 128, 128)
    hd = pl.pallas_call(
        _accum_body, out_shape=jax.ShapeDtypeStruct(flat, jnp.float32))(
        sdeg[0].reshape(flat), sdeg[1].reshape(flat), h.reshape(flat),
        temp.reshape(1, K + 1), *[p.reshape((2,) + flat) for p in parts])

    return pl.pallas_call(
        _softmax_body, out_shape=jax.ShapeDtypeStruct((N, C), jnp.float32))(
        hd.reshape(NP, C))
